# baseline (speedup 1.0000x reference)
import functools

import jax
import jax.numpy as jnp
from jax import lax
from jax.experimental import pallas as pl
from jax.experimental.pallas import tpu as pltpu
from jax.experimental.pallas import tpu_sc as plsc

B = 16384
D = 16
NC = 2
NS = 16
NW = NC * NS
BPW = B // NW
CHUNK = 128
NCHUNK = BPW // CHUNK
PACK = 128 // D


def _pack_body(in_ref, out_ref):
    x = in_ref[...]
    cb = x.shape[1]
    y = x.reshape(D, cb // PACK, PACK).transpose(1, 2, 0)
    out_ref[...] = y.reshape(cb // PACK, 128)


def _tc_pack(t, cb=4096):
    v = t.shape[1]
    rows = ((v // PACK + 7) // 8) * 8
    grid = (pl.cdiv(v, cb),)
    return pl.pallas_call(
        _pack_body,
        grid=grid,
        in_specs=[pl.BlockSpec((D, cb), lambda i: (0, i))],
        out_specs=pl.BlockSpec((cb // PACK, 128), lambda i: (i, 0)),
        out_shape=jax.ShapeDtypeStruct((rows, 128), jnp.float32),
    )(t)


def _sc_gather4(uidx3, iidx3, ue_gmf, ie_gmf, ue_mlp, ie_mlp):
    mesh = plsc.VectorSubcoreMesh(core_axis_name="c", subcore_axis_name="s")

    @functools.partial(
        pl.kernel,
        mesh=mesh,
        compiler_params=pltpu.CompilerParams(use_tc_tiling_on_sc=False),
        out_type=[jax.ShapeDtypeStruct((B, D), jnp.float32)] * 4,
        scratch_types=[
            pltpu.VMEM((NCHUNK, CHUNK), jnp.int32),
            pltpu.VMEM((NCHUNK, CHUNK), jnp.int32),
            pltpu.VMEM((BPW, D), jnp.float32),
            pltpu.VMEM((BPW, D), jnp.float32),
            pltpu.VMEM((BPW, D), jnp.float32),
            pltpu.VMEM((BPW, D), jnp.float32),
            pltpu.SemaphoreType.DMA,
        ],
    )
    def k(uidx_hbm, iidx_hbm, ug_hbm, ig_hbm, um_hbm, im_hbm,
          out_ug, out_ig, out_um, out_im,
          uidx_v, iidx_v, ug_v, ig_v, um_v, im_v, sem):
        wid = lax.axis_index("s") * NC + lax.axis_index("c")
        base = wid * BPW
        pltpu.sync_copy(uidx_hbm.at[wid], uidx_v)
        pltpu.sync_copy(iidx_hbm.at[wid], iidx_v)
        copies = []
        for j in range(NCHUNK):
            sl = pl.ds(j * CHUNK, CHUNK)
            copies.append(pltpu.async_copy(ug_hbm.at[uidx_v.at[j]], ug_v.at[sl], sem))
            copies.append(pltpu.async_copy(um_hbm.at[uidx_v.at[j]], um_v.at[sl], sem))
            copies.append(pltpu.async_copy(ig_hbm.at[iidx_v.at[j]], ig_v.at[sl], sem))
            copies.append(pltpu.async_copy(im_hbm.at[iidx_v.at[j]], im_v.at[sl], sem))
        for c in copies:
            c.wait()
        pltpu.sync_copy(ug_v, out_ug.at[pl.ds(base, BPW)])
        pltpu.sync_copy(ig_v, out_ig.at[pl.ds(base, BPW)])
        pltpu.sync_copy(um_v, out_um.at[pl.ds(base, BPW)])
        pltpu.sync_copy(im_v, out_im.at[pl.ds(base, BPW)])

    return k(uidx3, iidx3, ue_gmf, ie_gmf, ue_mlp, ie_mlp)


def _dot_t(x, w):
    return lax.dot_general(x, w, (((1,), (1,)), ((), ())),
                           preferred_element_type=jnp.float32)


def _tc_body(ug_ref, ig_ref, um_ref, im_ref,
             w0u_ref, w0i_ref, b0_ref, w1_ref, b1_ref, w2_ref, b2_ref,
             w3_ref, b3_ref, wpg_ref, wph_ref, bp_ref, out_ref):
    gmf = ug_ref[...] * ig_ref[...]
    h = _dot_t(um_ref[...], w0u_ref[...]) + _dot_t(im_ref[...], w0i_ref[...])
    h = jnp.maximum(h + b0_ref[...], 0.0)
    h = jnp.maximum(_dot_t(h, w1_ref[...]) + b1_ref[...], 0.0)
    h = jnp.maximum(_dot_t(h, w2_ref[...]) + b2_ref[...], 0.0)
    h = jnp.maximum(_dot_t(h, w3_ref[...]) + b3_ref[...], 0.0)
    pred = _dot_t(gmf, wpg_ref[...]) + _dot_t(h, wph_ref[...]) + bp_ref[...]
    out_ref[...] = pred


def _tc_mlp(ug, ig, um, im, W0u, W0i, b0, W1, b1, W2, b2, W3, b3,
            Wpg, Wph, bp2):
    BB = 2048
    grid = (B // BB,)
    row_spec = pl.BlockSpec((BB, D), lambda i: (i, 0))

    def full(a):
        return pl.BlockSpec(a.shape, lambda i: tuple(0 for _ in a.shape))

    return pl.pallas_call(
        _tc_body,
        grid=grid,
        in_specs=[row_spec, row_spec, row_spec, row_spec,
                  full(W0u), full(W0i), full(b0), full(W1), full(b1),
                  full(W2), full(b2), full(W3), full(b3),
                  full(Wpg), full(Wph), full(bp2)],
        out_specs=pl.BlockSpec((BB, 1), lambda i: (i, 0)),
        out_shape=jax.ShapeDtypeStruct((B, 1), jnp.float32),
    )(ug, ig, um, im, W0u, W0i, b0, W1, b1, W2, b2, W3, b3, Wpg, Wph, bp2)


def kernel(user_indices, item_indices, user_embed_gmf, item_embed_gmf,
           user_embed_mlp, item_embed_mlp,
           W0, b0, W1, b1, W2, b2, W3, b3, Wp, bp):
    uidx3 = user_indices.astype(jnp.int32).reshape(NW, NCHUNK, CHUNK)
    iidx3 = item_indices.astype(jnp.int32).reshape(NW, NCHUNK, CHUNK)
    def lin(t):
        p = _tc_pack(t.T)
        return p.reshape(p.shape[0] * PACK, D)

    ug_lin = lin(user_embed_gmf)
    um_lin = lin(user_embed_mlp)
    ig_lin = lin(item_embed_gmf)
    im_lin = lin(item_embed_mlp)
    ug, ig, um, im = _sc_gather4(uidx3, iidx3, ug_lin, ig_lin, um_lin, im_lin)
    W0u, W0i = W0[:, :D], W0[:, D:]
    Wpg, Wph = Wp[:, :D], Wp[:, D:]
    out = _tc_mlp(ug, ig, um, im, W0u, W0i, b0.reshape(1, -1),
                  W1, b1.reshape(1, -1), W2, b2.reshape(1, -1),
                  W3, b3.reshape(1, -1), Wpg, Wph, bp.reshape(1, 1))
    return out.reshape(B)


# speedup vs baseline: 2.1749x; 2.1749x over previous
import functools

import jax
import jax.numpy as jnp
from jax import lax
from jax.experimental import pallas as pl
from jax.experimental.pallas import tpu as pltpu
from jax.experimental.pallas import tpu_sc as plsc

B = 16384
D = 16
NC = 2
NS = 16
NW = NC * NS
BPW = B // NW
CHUNK = 128
NCHUNK = BPW // CHUNK
PACK = 128 // D


def _pack_body(in_ref, out_ref):
    x = in_ref[...]
    cb = x.shape[1]
    y = x.reshape(D, cb // PACK, PACK).transpose(1, 2, 0)
    out_ref[...] = y.reshape(cb // PACK, 128)


def _tc_pack(t, cb=4096):
    v = t.shape[1]
    rows = ((v // PACK + 7) // 8) * 8
    grid = (pl.cdiv(v, cb),)
    return pl.pallas_call(
        _pack_body,
        grid=grid,
        in_specs=[pl.BlockSpec((D, cb), lambda i: (0, i))],
        out_specs=pl.BlockSpec((cb // PACK, 128), lambda i: (i, 0)),
        out_shape=jax.ShapeDtypeStruct((rows, 128), jnp.float32),
    )(t)


def _sc_gather4(uidx3, iidx3, ue_gmf, ie_gmf, ue_mlp, ie_mlp):
    mesh = plsc.VectorSubcoreMesh(core_axis_name="c", subcore_axis_name="s")

    @functools.partial(
        pl.kernel,
        mesh=mesh,
        compiler_params=pltpu.CompilerParams(use_tc_tiling_on_sc=False),
        out_type=[jax.ShapeDtypeStruct((B, D), jnp.float32)] * 4,
        scratch_types=[
            pltpu.VMEM((NCHUNK, CHUNK), jnp.int32),
            pltpu.VMEM((NCHUNK, CHUNK), jnp.int32),
            pltpu.VMEM((BPW, D), jnp.float32),
            pltpu.VMEM((BPW, D), jnp.float32),
            pltpu.VMEM((BPW, D), jnp.float32),
            pltpu.VMEM((BPW, D), jnp.float32),
            pltpu.SemaphoreType.DMA,
        ],
    )
    def k(uidx_hbm, iidx_hbm, ug_hbm, ig_hbm, um_hbm, im_hbm,
          out_ug, out_ig, out_um, out_im,
          uidx_v, iidx_v, ug_v, ig_v, um_v, im_v, sem):
        wid = lax.axis_index("s") * NC + lax.axis_index("c")
        base = wid * BPW
        pltpu.sync_copy(uidx_hbm.at[wid], uidx_v)
        pltpu.sync_copy(iidx_hbm.at[wid], iidx_v)
        copies = []
        for j in range(NCHUNK):
            sl = pl.ds(j * CHUNK, CHUNK)
            copies.append(pltpu.async_copy(ug_hbm.at[uidx_v.at[j]], ug_v.at[sl], sem))
            copies.append(pltpu.async_copy(um_hbm.at[uidx_v.at[j]], um_v.at[sl], sem))
            copies.append(pltpu.async_copy(ig_hbm.at[iidx_v.at[j]], ig_v.at[sl], sem))
            copies.append(pltpu.async_copy(im_hbm.at[iidx_v.at[j]], im_v.at[sl], sem))
        for c in copies:
            c.wait()
        pltpu.sync_copy(ug_v, out_ug.at[pl.ds(base, BPW)])
        pltpu.sync_copy(ig_v, out_ig.at[pl.ds(base, BPW)])
        pltpu.sync_copy(um_v, out_um.at[pl.ds(base, BPW)])
        pltpu.sync_copy(im_v, out_im.at[pl.ds(base, BPW)])

    return k(uidx3, iidx3, ue_gmf, ie_gmf, ue_mlp, ie_mlp)


def _dot_t(x, w):
    return lax.dot_general(x, w, (((1,), (1,)), ((), ())),
                           preferred_element_type=jnp.float32)


def _tc_body(ug_ref, ig_ref, um_ref, im_ref,
             w0u_ref, w0i_ref, b0_ref, w1_ref, b1_ref, w2_ref, b2_ref,
             w3_ref, b3_ref, wpg_ref, wph_ref, bp_ref, out_ref):
    gmf = ug_ref[...] * ig_ref[...]
    h = _dot_t(um_ref[...], w0u_ref[...]) + _dot_t(im_ref[...], w0i_ref[...])
    h = jnp.maximum(h + b0_ref[...], 0.0)
    h = jnp.maximum(_dot_t(h, w1_ref[...]) + b1_ref[...], 0.0)
    h = jnp.maximum(_dot_t(h, w2_ref[...]) + b2_ref[...], 0.0)
    h = jnp.maximum(_dot_t(h, w3_ref[...]) + b3_ref[...], 0.0)
    pred = _dot_t(gmf, wpg_ref[...]) + _dot_t(h, wph_ref[...]) + bp_ref[...]
    out_ref[...] = pred


def _tc_mlp(ug, ig, um, im, W0u, W0i, b0, W1, b1, W2, b2, W3, b3,
            Wpg, Wph, bp2):
    BB = 2048
    grid = (B // BB,)
    row_spec = pl.BlockSpec((BB, D), lambda i: (i, 0))

    def full(a):
        return pl.BlockSpec(a.shape, lambda i: tuple(0 for _ in a.shape))

    return pl.pallas_call(
        _tc_body,
        grid=grid,
        in_specs=[row_spec, row_spec, row_spec, row_spec,
                  full(W0u), full(W0i), full(b0), full(W1), full(b1),
                  full(W2), full(b2), full(W3), full(b3),
                  full(Wpg), full(Wph), full(bp2)],
        out_specs=pl.BlockSpec((BB, 1), lambda i: (i, 0)),
        out_shape=jax.ShapeDtypeStruct((B, 1), jnp.float32),
    )(ug, ig, um, im, W0u, W0i, b0, W1, b1, W2, b2, W3, b3, Wpg, Wph, bp2)


def kernel(user_indices, item_indices, user_embed_gmf, item_embed_gmf,
           user_embed_mlp, item_embed_mlp,
           W0, b0, W1, b1, W2, b2, W3, b3, Wp, bp):
    uidx3 = user_indices.astype(jnp.int32).reshape(NW, NCHUNK, CHUNK)
    iidx3 = item_indices.astype(jnp.int32).reshape(NW, NCHUNK, CHUNK)
    def lin(t):
        p = lax.optimization_barrier(t.reshape(t.shape[0] // PACK, 128))
        return p.reshape(t.shape[0], D)

    ug_lin = lin(user_embed_gmf)
    um_lin = lin(user_embed_mlp)
    ig_lin = lin(item_embed_gmf)
    im_lin = lin(item_embed_mlp)
    ug, ig, um, im = _sc_gather4(uidx3, iidx3, ug_lin, ig_lin, um_lin, im_lin)
    W0u, W0i = W0[:, :D], W0[:, D:]
    Wpg, Wph = Wp[:, :D], Wp[:, D:]
    out = _tc_mlp(ug, ig, um, im, W0u, W0i, b0.reshape(1, -1),
                  W1, b1.reshape(1, -1), W2, b2.reshape(1, -1),
                  W3, b3.reshape(1, -1), Wpg, Wph, bp.reshape(1, 1))
    return out.reshape(B)


# speedup vs baseline: 2.1768x; 1.0009x over previous
import functools

import jax
import jax.numpy as jnp
from jax import lax
from jax.experimental import pallas as pl
from jax.experimental.pallas import tpu as pltpu
from jax.experimental.pallas import tpu_sc as plsc

B = 16384
D = 16
NC = 2
NS = 16
NW = NC * NS
BPW = B // NW
CHUNK = 128
NCHUNK = BPW // CHUNK
PACK = 128 // D


def _sc_gather4(uidx3, iidx3, ue_gmf, ie_gmf, ue_mlp, ie_mlp):
    mesh = plsc.VectorSubcoreMesh(core_axis_name="c", subcore_axis_name="s")

    @functools.partial(
        pl.kernel,
        mesh=mesh,
        compiler_params=pltpu.CompilerParams(use_tc_tiling_on_sc=False),
        out_type=[jax.ShapeDtypeStruct((B, D), jnp.float32)] * 4,
        scratch_types=[
            pltpu.VMEM((NCHUNK, CHUNK), jnp.int32),
            pltpu.VMEM((NCHUNK, CHUNK), jnp.int32),
            pltpu.VMEM((BPW, D), jnp.float32),
            pltpu.VMEM((BPW, D), jnp.float32),
            pltpu.VMEM((BPW, D), jnp.float32),
            pltpu.VMEM((BPW, D), jnp.float32),
            pltpu.SemaphoreType.DMA,
        ],
    )
    def k(uidx_hbm, iidx_hbm, ug_hbm, ig_hbm, um_hbm, im_hbm,
          out_ug, out_ig, out_um, out_im,
          uidx_v, iidx_v, ug_v, ig_v, um_v, im_v, sem):
        wid = lax.axis_index("s") * NC + lax.axis_index("c")
        base = wid * BPW
        pltpu.sync_copy(uidx_hbm.at[wid], uidx_v)
        pltpu.sync_copy(iidx_hbm.at[wid], iidx_v)
        copies = []
        for j in range(NCHUNK):
            sl = pl.ds(j * CHUNK, CHUNK)
            copies.append(pltpu.async_copy(ug_hbm.at[uidx_v.at[j]], ug_v.at[sl], sem))
            copies.append(pltpu.async_copy(um_hbm.at[uidx_v.at[j]], um_v.at[sl], sem))
            copies.append(pltpu.async_copy(ig_hbm.at[iidx_v.at[j]], ig_v.at[sl], sem))
            copies.append(pltpu.async_copy(im_hbm.at[iidx_v.at[j]], im_v.at[sl], sem))
        for c in copies:
            c.wait()
        pltpu.sync_copy(ug_v, out_ug.at[pl.ds(base, BPW)])
        pltpu.sync_copy(ig_v, out_ig.at[pl.ds(base, BPW)])
        pltpu.sync_copy(um_v, out_um.at[pl.ds(base, BPW)])
        pltpu.sync_copy(im_v, out_im.at[pl.ds(base, BPW)])

    return k(uidx3, iidx3, ue_gmf, ie_gmf, ue_mlp, ie_mlp)


def _dot_t(x, w):
    return lax.dot_general(x, w, (((1,), (1,)), ((), ())),
                           preferred_element_type=jnp.float32)


def _tc_body(ug_ref, ig_ref, um_ref, im_ref,
             w0u_ref, w0i_ref, b0_ref, w1_ref, b1_ref, w2_ref, b2_ref,
             w3_ref, b3_ref, wpg_ref, wph_ref, bp_ref, out_ref):
    gmf = ug_ref[...] * ig_ref[...]
    h = _dot_t(um_ref[...], w0u_ref[...]) + _dot_t(im_ref[...], w0i_ref[...])
    h = jnp.maximum(h + b0_ref[...], 0.0)
    h = jnp.maximum(_dot_t(h, w1_ref[...]) + b1_ref[...], 0.0)
    h = jnp.maximum(_dot_t(h, w2_ref[...]) + b2_ref[...], 0.0)
    h = jnp.maximum(_dot_t(h, w3_ref[...]) + b3_ref[...], 0.0)
    pred = _dot_t(gmf, wpg_ref[...]) + _dot_t(h, wph_ref[...]) + bp_ref[...]
    out_ref[...] = pred


def _tc_mlp(ug, ig, um, im, W0u, W0i, b0, W1, b1, W2, b2, W3, b3,
            Wpg, Wph, bp2):
    BB = 2048
    grid = (B // BB,)
    row_spec = pl.BlockSpec((BB, D), lambda i: (i, 0))

    def full(a):
        return pl.BlockSpec(a.shape, lambda i: tuple(0 for _ in a.shape))

    return pl.pallas_call(
        _tc_body,
        grid=grid,
        in_specs=[row_spec, row_spec, row_spec, row_spec,
                  full(W0u), full(W0i), full(b0), full(W1), full(b1),
                  full(W2), full(b2), full(W3), full(b3),
                  full(Wpg), full(Wph), full(bp2)],
        out_specs=pl.BlockSpec((BB, 1), lambda i: (i, 0)),
        out_shape=jax.ShapeDtypeStruct((B, 1), jnp.float32),
    )(ug, ig, um, im, W0u, W0i, b0, W1, b1, W2, b2, W3, b3, Wpg, Wph, bp2)


def kernel(user_indices, item_indices, user_embed_gmf, item_embed_gmf,
           user_embed_mlp, item_embed_mlp,
           W0, b0, W1, b1, W2, b2, W3, b3, Wp, bp):
    uidx3 = user_indices.astype(jnp.int32).reshape(NW, NCHUNK, CHUNK)
    iidx3 = item_indices.astype(jnp.int32).reshape(NW, NCHUNK, CHUNK)
    def lin(t):
        p = lax.optimization_barrier(t.reshape(t.shape[0] // PACK, 128))
        return p.reshape(t.shape[0], D)

    ug_lin = lin(user_embed_gmf)
    um_lin = lin(user_embed_mlp)
    ig_lin = lin(item_embed_gmf)
    im_lin = lin(item_embed_mlp)
    ug, ig, um, im = _sc_gather4(uidx3, iidx3, ug_lin, ig_lin, um_lin, im_lin)
    W0u, W0i = W0[:, :D], W0[:, D:]
    Wpg, Wph = Wp[:, :D], Wp[:, D:]
    out = _tc_mlp(ug, ig, um, im, W0u, W0i, b0.reshape(1, -1),
                  W1, b1.reshape(1, -1), W2, b2.reshape(1, -1),
                  W3, b3.reshape(1, -1), Wpg, Wph, bp.reshape(1, 1))
    return out.reshape(B)
